# Initial kernel scaffold; baseline (speedup 1.0000x reference)
import jax
import jax.numpy as jnp
from jax.experimental import pallas as pl


def kernel(x, table):
    raise NotImplementedError("write your pallas kernel here")


# speedup vs baseline: 4.8011x; 4.8011x over previous
import functools

import jax
import jax.numpy as jnp
from jax import lax
from jax.experimental import pallas as pl
from jax.experimental.pallas import tpu as pltpu
from jax.experimental.pallas import tpu_sc as plsc

_VOCAB = 1_000_000
_EMB = 32
_BATCH = 16384
_HIST = 200
_B = _BATCH * _HIST

_NC = 2
_NS = 16
_NW = _NC * _NS
_B_PER_W = _B // _NW
_CHUNK = 1024
_N_CHUNKS = _B_PER_W // _CHUNK


@functools.partial(
    pl.kernel,
    mesh=plsc.VectorSubcoreMesh(core_axis_name="c", subcore_axis_name="s"),
    out_type=jax.ShapeDtypeStruct((_B, _EMB), jnp.float32),
    scratch_types=[
        pltpu.VMEM((_CHUNK,), jnp.int32),
        pltpu.VMEM((_CHUNK, _EMB), jnp.float32),
        pltpu.SemaphoreType.DMA,
    ],
    compiler_params=pltpu.CompilerParams(use_tc_tiling_on_sc=False),
)
def _gather_kernel(idx_hbm, table_hbm, out_hbm, idx_v, rows_v, sem):
    wid = lax.axis_index("s") * _NC + lax.axis_index("c")
    base_w = wid * _B_PER_W

    def body(i, carry):
        base = pl.multiple_of(base_w + i * _CHUNK, _CHUNK)
        pltpu.sync_copy(idx_hbm.at[pl.ds(base, _CHUNK)], idx_v)
        pltpu.async_copy(table_hbm.at[idx_v], rows_v, sem).wait()
        pltpu.sync_copy(rows_v, out_hbm.at[pl.ds(base, _CHUNK)])
        return carry

    lax.fori_loop(0, _N_CHUNKS, body, 0)


def kernel(x, table):
    flat = x.reshape(_B)
    out = _gather_kernel(flat, table)
    return out.reshape(_BATCH, _HIST, _EMB)


# speedup vs baseline: 5.0429x; 1.0504x over previous
import functools

import jax
import jax.numpy as jnp
from jax import lax
from jax.experimental import pallas as pl
from jax.experimental.pallas import tpu as pltpu
from jax.experimental.pallas import tpu_sc as plsc

_VOCAB = 1_000_000
_EMB = 32
_BATCH = 16384
_HIST = 200
_B = _BATCH * _HIST

_NC = 2
_NS = 16
_NW = _NC * _NS
_B_PER_W = _B // _NW
_CHUNK = 1024
_N_CHUNKS = _B_PER_W // _CHUNK


@functools.partial(
    pl.kernel,
    mesh=plsc.VectorSubcoreMesh(core_axis_name="c", subcore_axis_name="s"),
    out_type=jax.ShapeDtypeStruct((_B, _EMB), jnp.float32),
    scratch_types=[
        pltpu.VMEM((2, _CHUNK), jnp.int32),
        pltpu.VMEM((2, _CHUNK, _EMB), jnp.float32),
        pltpu.SemaphoreType.DMA((2,)),
        pltpu.SemaphoreType.DMA((2,)),
        pltpu.SemaphoreType.DMA((2,)),
    ],
    compiler_params=pltpu.CompilerParams(use_tc_tiling_on_sc=False),
)
def _gather_kernel(idx_hbm, table_hbm, out_hbm, idx_v, rows_v,
                   idx_sem, gat_sem, out_sem):
    wid = lax.axis_index("s") * _NC + lax.axis_index("c")
    base_w = wid * _B_PER_W

    def cbase(i):
        return pl.multiple_of(base_w + i * _CHUNK, 8)

    def start_idx(i, b):
        pltpu.async_copy(idx_hbm.at[pl.ds(cbase(i), _CHUNK)], idx_v.at[b],
                         idx_sem.at[b])

    def wait_idx(i, b):
        pltpu.make_async_copy(idx_hbm.at[pl.ds(cbase(i), _CHUNK)],
                              idx_v.at[b], idx_sem.at[b]).wait()

    def start_gather(b):
        pltpu.async_copy(table_hbm.at[idx_v.at[b]], rows_v.at[b],
                         gat_sem.at[b])

    def wait_gather(b):
        pltpu.make_async_copy(table_hbm.at[idx_v.at[b]], rows_v.at[b],
                              gat_sem.at[b]).wait()

    def start_out(i, b):
        pltpu.async_copy(rows_v.at[b], out_hbm.at[pl.ds(cbase(i), _CHUNK)],
                         out_sem.at[b])

    def wait_out(i, b):
        pltpu.make_async_copy(rows_v.at[b],
                              out_hbm.at[pl.ds(cbase(i), _CHUNK)],
                              out_sem.at[b]).wait()

    start_idx(0, 0)
    wait_idx(0, 0)
    start_gather(0)
    start_idx(1, 1)

    def body(i, carry):
        b = lax.rem(i, 2)
        bp = 1 - b
        wait_idx(i, b)

        @pl.when(i >= 2)
        def _():
            wait_out(i - 2, b)

        start_gather(b)
        wait_gather(bp)
        start_out(i - 1, bp)

        @pl.when(i + 1 < _N_CHUNKS)
        def _():
            start_idx(i + 1, bp)

        return carry

    lax.fori_loop(1, _N_CHUNKS, body, 0)

    bl = (_N_CHUNKS - 1) % 2
    wait_gather(bl)
    start_out(_N_CHUNKS - 1, bl)
    wait_out(_N_CHUNKS - 2, 1 - bl)
    wait_out(_N_CHUNKS - 1, bl)


def kernel(x, table):
    flat = x.reshape(_B)
    out = _gather_kernel(flat, table)
    return out.reshape(_BATCH, _HIST, _EMB)


# speedup vs baseline: 5.0438x; 1.0002x over previous
import functools

import jax
import jax.numpy as jnp
from jax import lax
from jax.experimental import pallas as pl
from jax.experimental.pallas import tpu as pltpu
from jax.experimental.pallas import tpu_sc as plsc

_VOCAB = 1_000_000
_EMB = 32
_BATCH = 16384
_HIST = 200
_B = _BATCH * _HIST

_NC = 2
_NS = 16
_NW = _NC * _NS
_B_PER_W = _B // _NW
_CHUNK = 1024
_N_CHUNKS = _B_PER_W // _CHUNK

_NBUF = 3
_GLAG = 2
_NIDX = 6


@functools.partial(
    pl.kernel,
    mesh=plsc.VectorSubcoreMesh(core_axis_name="c", subcore_axis_name="s"),
    out_type=jax.ShapeDtypeStruct((_B, _EMB), jnp.float32),
    scratch_types=[
        pltpu.VMEM((_NIDX, _CHUNK), jnp.int32),
        pltpu.VMEM((_NBUF, _CHUNK, _EMB), jnp.float32),
        pltpu.SemaphoreType.DMA((_NIDX,)),
        pltpu.SemaphoreType.DMA((_NBUF,)),
        pltpu.SemaphoreType.DMA((_NBUF,)),
    ],
    compiler_params=pltpu.CompilerParams(use_tc_tiling_on_sc=False),
)
def _gather_kernel(idx_hbm, table_hbm, out_hbm, idx_v, rows_v,
                   idx_sem, gat_sem, out_sem):
    wid = lax.axis_index("s") * _NC + lax.axis_index("c")
    base_w = wid * _B_PER_W

    def cbase(i):
        return pl.multiple_of(base_w + i * _CHUNK, 8)

    def start_idx(i):
        b = lax.rem(i, _NIDX)
        pltpu.async_copy(idx_hbm.at[pl.ds(cbase(i), _CHUNK)], idx_v.at[b],
                         idx_sem.at[b])

    def wait_idx(i):
        b = lax.rem(i, _NIDX)
        pltpu.make_async_copy(idx_hbm.at[pl.ds(cbase(i), _CHUNK)],
                              idx_v.at[b], idx_sem.at[b]).wait()

    def start_gather(i):
        b = lax.rem(i, _NBUF)
        pltpu.async_copy(table_hbm.at[idx_v.at[lax.rem(i, _NIDX)]],
                         rows_v.at[b], gat_sem.at[b])

    def wait_gather(i):
        b = lax.rem(i, _NBUF)
        pltpu.make_async_copy(table_hbm.at[idx_v.at[lax.rem(i, _NIDX)]],
                              rows_v.at[b], gat_sem.at[b]).wait()

    def start_out(i):
        b = lax.rem(i, _NBUF)
        pltpu.async_copy(rows_v.at[b], out_hbm.at[pl.ds(cbase(i), _CHUNK)],
                         out_sem.at[b])

    def wait_out(i):
        b = lax.rem(i, _NBUF)
        pltpu.make_async_copy(rows_v.at[b],
                              out_hbm.at[pl.ds(cbase(i), _CHUNK)],
                              out_sem.at[b]).wait()

    for p in range(min(_NIDX, _N_CHUNKS)):
        start_idx(p)

    def body(i, carry):
        @pl.when(i < _N_CHUNKS)
        def _():
            wait_idx(i)

            @pl.when(i >= _NBUF)
            def _():
                wait_out(i - _NBUF)

            start_gather(i)

        j = i - _GLAG

        @pl.when(j >= 0)
        def _():
            wait_gather(j)
            start_out(j)

            @pl.when(j + _NIDX < _N_CHUNKS)
            def _():
                start_idx(j + _NIDX)

        return carry

    lax.fori_loop(0, _N_CHUNKS + _GLAG, body, 0)

    def drain(i, carry):
        wait_out(i)
        return carry

    lax.fori_loop(_N_CHUNKS - _NBUF, _N_CHUNKS, drain, 0)


def kernel(x, table):
    flat = x.reshape(_B)
    out = _gather_kernel(flat, table)
    return out.reshape(_BATCH, _HIST, _EMB)


# speedup vs baseline: 5.0471x; 1.0007x over previous
import functools

import jax
import jax.numpy as jnp
from jax import lax
from jax.experimental import pallas as pl
from jax.experimental.pallas import tpu as pltpu
from jax.experimental.pallas import tpu_sc as plsc

_VOCAB = 1_000_000
_EMB = 32
_BATCH = 16384
_HIST = 200
_B = _BATCH * _HIST

_NC = 2
_NS = 16
_NW = _NC * _NS
_B_PER_W = _B // _NW
_CHUNK = 512
_N_CHUNKS = _B_PER_W // _CHUNK

_NBUF = 6
_GLAG = 4
_NIDX = 12


@functools.partial(
    pl.kernel,
    mesh=plsc.VectorSubcoreMesh(core_axis_name="c", subcore_axis_name="s"),
    out_type=jax.ShapeDtypeStruct((_B, _EMB), jnp.float32),
    scratch_types=[
        pltpu.VMEM((_NIDX, _CHUNK), jnp.int32),
        pltpu.VMEM((_NBUF, _CHUNK, _EMB), jnp.float32),
        pltpu.SemaphoreType.DMA((_NIDX,)),
        pltpu.SemaphoreType.DMA((_NBUF,)),
        pltpu.SemaphoreType.DMA((_NBUF,)),
    ],
    compiler_params=pltpu.CompilerParams(use_tc_tiling_on_sc=False),
)
def _gather_kernel(idx_hbm, table_hbm, out_hbm, idx_v, rows_v,
                   idx_sem, gat_sem, out_sem):
    wid = lax.axis_index("s") * _NC + lax.axis_index("c")
    base_w = wid * _B_PER_W

    def cbase(i):
        return pl.multiple_of(base_w + i * _CHUNK, 8)

    def start_idx(i):
        b = lax.rem(i, _NIDX)
        pltpu.async_copy(idx_hbm.at[pl.ds(cbase(i), _CHUNK)], idx_v.at[b],
                         idx_sem.at[b])

    def wait_idx(i):
        b = lax.rem(i, _NIDX)
        pltpu.make_async_copy(idx_hbm.at[pl.ds(cbase(i), _CHUNK)],
                              idx_v.at[b], idx_sem.at[b]).wait()

    def start_gather(i):
        b = lax.rem(i, _NBUF)
        pltpu.async_copy(table_hbm.at[idx_v.at[lax.rem(i, _NIDX)]],
                         rows_v.at[b], gat_sem.at[b])

    def wait_gather(i):
        b = lax.rem(i, _NBUF)
        pltpu.make_async_copy(table_hbm.at[idx_v.at[lax.rem(i, _NIDX)]],
                              rows_v.at[b], gat_sem.at[b]).wait()

    def start_out(i):
        b = lax.rem(i, _NBUF)
        pltpu.async_copy(rows_v.at[b], out_hbm.at[pl.ds(cbase(i), _CHUNK)],
                         out_sem.at[b])

    def wait_out(i):
        b = lax.rem(i, _NBUF)
        pltpu.make_async_copy(rows_v.at[b],
                              out_hbm.at[pl.ds(cbase(i), _CHUNK)],
                              out_sem.at[b]).wait()

    for p in range(min(_NIDX, _N_CHUNKS)):
        start_idx(p)

    def body(i, carry):
        @pl.when(i < _N_CHUNKS)
        def _():
            wait_idx(i)

            @pl.when(i >= _NBUF)
            def _():
                wait_out(i - _NBUF)

            start_gather(i)

        j = i - _GLAG

        @pl.when(j >= 0)
        def _():
            wait_gather(j)
            start_out(j)

            @pl.when(j + _NIDX < _N_CHUNKS)
            def _():
                start_idx(j + _NIDX)

        return carry

    lax.fori_loop(0, _N_CHUNKS + _GLAG, body, 0)

    def drain(i, carry):
        wait_out(i)
        return carry

    lax.fori_loop(_N_CHUNKS - _NBUF, _N_CHUNKS, drain, 0)


def kernel(x, table):
    flat = x.reshape(_B)
    out = _gather_kernel(flat, table)
    return out.reshape(_BATCH, _HIST, _EMB)


# speedup vs baseline: 5.0482x; 1.0002x over previous
import functools

import jax
import jax.numpy as jnp
from jax import lax
from jax.experimental import pallas as pl
from jax.experimental.pallas import tpu as pltpu
from jax.experimental.pallas import tpu_sc as plsc

_VOCAB = 1_000_000
_EMB = 32
_BATCH = 16384
_HIST = 200
_B = _BATCH * _HIST

_NC = 2
_NS = 16
_NW = _NC * _NS
_ROWS_PER_W = _BATCH // _NW

_R = 4
_CHUNK = _R * _HIST
_N_CHUNKS = _ROWS_PER_W // _R

_NBUF = 3
_GLAG = 2
_NIDX = 6


@functools.partial(
    pl.kernel,
    mesh=plsc.VectorSubcoreMesh(core_axis_name="c", subcore_axis_name="s"),
    out_type=jax.ShapeDtypeStruct((_BATCH, _HIST, _EMB), jnp.float32),
    scratch_types=[
        pltpu.VMEM((_NIDX, _CHUNK), jnp.int32),
        pltpu.VMEM((_NBUF, _CHUNK, _EMB), jnp.float32),
        pltpu.SemaphoreType.DMA((_NIDX,)),
        pltpu.SemaphoreType.DMA((_NBUF,)),
        pltpu.SemaphoreType.DMA((_NBUF,)),
    ],
    compiler_params=pltpu.CompilerParams(use_tc_tiling_on_sc=False),
)
def _gather_kernel(idx_hbm, table_hbm, out_hbm, idx_v, rows_v,
                   idx_sem, gat_sem, out_sem):
    wid = lax.axis_index("s") * _NC + lax.axis_index("c")
    row_w = wid * _ROWS_PER_W

    def cbase(i):
        return pl.multiple_of((row_w + i * _R) * _HIST, 8)

    def crow(i):
        return row_w + i * _R

    def start_idx(i):
        b = lax.rem(i, _NIDX)
        pltpu.async_copy(idx_hbm.at[pl.ds(cbase(i), _CHUNK)], idx_v.at[b],
                         idx_sem.at[b])

    def wait_idx(i):
        b = lax.rem(i, _NIDX)
        pltpu.make_async_copy(idx_hbm.at[pl.ds(cbase(i), _CHUNK)],
                              idx_v.at[b], idx_sem.at[b]).wait()

    def start_gather(i):
        b = lax.rem(i, _NBUF)
        pltpu.async_copy(table_hbm.at[idx_v.at[lax.rem(i, _NIDX)]],
                         rows_v.at[b], gat_sem.at[b])

    def wait_gather(i):
        b = lax.rem(i, _NBUF)
        pltpu.make_async_copy(table_hbm.at[idx_v.at[lax.rem(i, _NIDX)]],
                              rows_v.at[b], gat_sem.at[b]).wait()

    def start_out(i):
        b = lax.rem(i, _NBUF)
        for r in range(_R):
            pltpu.async_copy(rows_v.at[b, pl.ds(r * _HIST, _HIST)],
                             out_hbm.at[crow(i) + r], out_sem.at[b])

    def wait_out(i):
        b = lax.rem(i, _NBUF)
        for r in range(_R):
            pltpu.make_async_copy(rows_v.at[b, pl.ds(r * _HIST, _HIST)],
                                  out_hbm.at[crow(i) + r],
                                  out_sem.at[b]).wait()

    for p in range(min(_NIDX, _N_CHUNKS)):
        start_idx(p)

    def body(i, carry):
        @pl.when(i < _N_CHUNKS)
        def _():
            wait_idx(i)

            @pl.when(i >= _NBUF)
            def _():
                wait_out(i - _NBUF)

            start_gather(i)

        j = i - _GLAG

        @pl.when(j >= 0)
        def _():
            wait_gather(j)
            start_out(j)

            @pl.when(j + _NIDX < _N_CHUNKS)
            def _():
                start_idx(j + _NIDX)

        return carry

    lax.fori_loop(0, _N_CHUNKS + _GLAG, body, 0)

    def drain(i, carry):
        wait_out(i)
        return carry

    lax.fori_loop(_N_CHUNKS - _NBUF, _N_CHUNKS, drain, 0)


def kernel(x, table):
    flat = x.reshape(_B)
    return _gather_kernel(flat, table)


# speedup vs baseline: 6.0179x; 1.1921x over previous
import functools

import jax
import jax.numpy as jnp
from jax import lax
from jax.experimental import pallas as pl
from jax.experimental.pallas import tpu as pltpu
from jax.experimental.pallas import tpu_sc as plsc

_VOCAB = 1_000_000
_EMB = 32
_BATCH = 16384
_HIST = 200
_B = _BATCH * _HIST

_NC = 2
_NS = 16
_NW = _NC * _NS

_CHUNK = 800
_QROWS = _CHUNK // 4
_N_CHUNKS = _B // (_CHUNK * _NW)

_NBUF = 3
_GLAG = 2
_NIDX = 6
_NB2 = 4


@functools.partial(
    pl.kernel,
    mesh=plsc.VectorSubcoreMesh(core_axis_name="c", subcore_axis_name="s"),
    out_type=jax.ShapeDtypeStruct((_B // 4, 128), jnp.float32),
    scratch_types=[
        pltpu.VMEM((_NIDX, _CHUNK), jnp.int32),
        pltpu.VMEM((_NBUF, 4, _QROWS, _EMB), jnp.float32),
        pltpu.SemaphoreType.DMA((_NIDX,)),
        pltpu.SemaphoreType.DMA((_NBUF,)),
        pltpu.SemaphoreType.DMA((_NBUF,)),
    ],
    compiler_params=pltpu.CompilerParams(use_tc_tiling_on_sc=False),
)
def _gather_kernel(idx_hbm, table_hbm, y_hbm, idx_v, stage_v,
                   idx_sem, gat_sem, out_sem):
    wid = lax.axis_index("s") * _NC + lax.axis_index("c")

    def chunk_id(i):
        return wid * _N_CHUNKS + i

    def start_idx(i):
        b = lax.rem(i, _NIDX)
        base = pl.multiple_of(chunk_id(i) * _CHUNK, 8)
        pltpu.async_copy(idx_hbm.at[pl.ds(base, _CHUNK)], idx_v.at[b],
                         idx_sem.at[b])

    def wait_idx(i):
        b = lax.rem(i, _NIDX)
        base = pl.multiple_of(chunk_id(i) * _CHUNK, 8)
        pltpu.make_async_copy(idx_hbm.at[pl.ds(base, _CHUNK)],
                              idx_v.at[b], idx_sem.at[b]).wait()

    def gather_parts(i):
        b = lax.rem(i, _NBUF)
        n = lax.rem(i, _NIDX)
        for p in range(4):
            yield (table_hbm.at[idx_v.at[n, pl.ds(p * _QROWS, _QROWS)]],
                   stage_v.at[b, p])

    def start_gather(i):
        b = lax.rem(i, _NBUF)
        for src, dst in gather_parts(i):
            pltpu.async_copy(src, dst, gat_sem.at[b])

    def wait_gather(i):
        b = lax.rem(i, _NBUF)
        for src, dst in gather_parts(i):
            pltpu.make_async_copy(src, dst, gat_sem.at[b]).wait()

    def yout_parts(i):
        b = lax.rem(i, _NBUF)
        qbase = pl.multiple_of(chunk_id(i) * _QROWS, 8)
        for p in range(4):
            yield (stage_v.at[b, p],
                   y_hbm.at[pl.ds(qbase, _QROWS), pl.ds(p * _EMB, _EMB)])

    def start_out(i):
        b = lax.rem(i, _NBUF)
        for src, dst in yout_parts(i):
            pltpu.async_copy(src, dst, out_sem.at[b])

    def wait_out(i):
        b = lax.rem(i, _NBUF)
        for src, dst in yout_parts(i):
            pltpu.make_async_copy(src, dst, out_sem.at[b]).wait()

    for p in range(min(_NIDX, _N_CHUNKS)):
        start_idx(p)

    def body(i, carry):
        @pl.when(i < _N_CHUNKS)
        def _():
            wait_idx(i)

            @pl.when(i >= _NBUF)
            def _():
                wait_out(i - _NBUF)

            start_gather(i)

        j = i - _GLAG

        @pl.when(j >= 0)
        def _():
            wait_gather(j)
            start_out(j)

            @pl.when(j + _NIDX < _N_CHUNKS)
            def _():
                start_idx(j + _NIDX)

        return carry

    lax.fori_loop(0, _N_CHUNKS + _GLAG, body, 0)

    def drain(i, carry):
        wait_out(i)
        return carry

    lax.fori_loop(_N_CHUNKS - _NBUF, _N_CHUNKS, drain, 0)


_S = 16
_GC = _B // (_CHUNK * _S)


def _format_body(y_ref, o_ref):
    y3 = y_ref[...].reshape(_S, _QROWS, 128)
    for p in range(4):
        o_ref[:, p, :, :] = y3[:, :, p * _EMB:(p + 1) * _EMB]


_format_kernel = pl.pallas_call(
    _format_body,
    grid=(_GC,),
    in_specs=[pl.BlockSpec((_QROWS * _S, 128), lambda c: (c, 0))],
    out_specs=pl.BlockSpec((_S, 4, _QROWS, _EMB), lambda c: (c, 0, 0, 0)),
    out_shape=jax.ShapeDtypeStruct((_B // _CHUNK, 4, _QROWS, _EMB),
                                   jnp.float32),
)


def kernel(x, table):
    flat = x.reshape(_B)
    y = _gather_kernel(flat, table)
    out4 = _format_kernel(y)
    return out4.reshape(_BATCH, _HIST, _EMB)
